# Initial kernel scaffold; baseline (speedup 1.0000x reference)
import jax
import jax.numpy as jnp
from jax.experimental import pallas as pl


def kernel(x, edge_index, W_enc, b_enc, Wr0, Wn0, b0, Wr1, Wn1, b1, Wr2, Wn2, b2, W_dec, b_dec):
    raise NotImplementedError("write your pallas kernel here")


# speedup vs baseline: 11.5174x; 11.5174x over previous
import functools

import jax
import jax.numpy as jnp
from jax import lax
from jax.experimental import pallas as pl
from jax.experimental.pallas import tpu as pltpu
from jax.experimental.pallas import tpu_sc as plsc

N = 10000
NPAD = 10240
D = 128
E = 320000
NW = 32
CH = 128
NCH = 80
EPT = NCH * CH
EPAD = NW * EPT
STRIPE = NPAD // 16
WIN = 16
BLK = 1280
GRID = NPAD // BLK

_mesh = plsc.VectorSubcoreMesh(core_axis_name="c", subcore_axis_name="s")


def _wid():
    return lax.axis_index("s") * 2 + lax.axis_index("c")


@functools.partial(
    pl.kernel,
    out_type=jax.ShapeDtypeStruct((2, NPAD), jnp.float32),
    mesh=_mesh,
    scratch_types=[
        pltpu.VMEM((WIN, CH), jnp.int32),
        pltpu.VMEM((CH,), jnp.float32),
        pltpu.VMEM((STRIPE,), jnp.float32),
        pltpu.VMEM_SHARED((NPAD,), jnp.float32),
    ],
)
def _deg_kernel(dst_hbm, out_hbm, idx_v, ones_v, zero_v, acc):
    cid = lax.axis_index("c")
    sid = lax.axis_index("s")
    wid = _wid()

    def fill_ones(i, carry):
        ones_v[pl.ds(i * 16, 16)] = jnp.ones((16,), jnp.float32)
        return carry

    lax.fori_loop(0, CH // 16, fill_ones, 0)

    def fill_zero(i, carry):
        zero_v[pl.ds(i * 16, 16)] = jnp.zeros((16,), jnp.float32)
        return carry

    lax.fori_loop(0, STRIPE // 16, fill_zero, 0)
    pltpu.sync_copy(zero_v, acc.at[pl.ds(sid * STRIPE, STRIPE)])
    plsc.subcore_barrier()

    def window(w, carry):
        pltpu.sync_copy(dst_hbm.at[wid, pl.ds(w * WIN, WIN)], idx_v)

        def chunk(j, carry2):
            pltpu.sync_copy(ones_v, acc.at[idx_v.at[j]], add=True)
            return carry2

        lax.fori_loop(0, WIN, chunk, 0)
        return carry

    lax.fori_loop(0, NCH // WIN, window, 0)
    plsc.subcore_barrier()
    pltpu.sync_copy(acc.at[pl.ds(sid * STRIPE, STRIPE)],
                    out_hbm.at[cid, pl.ds(sid * STRIPE, STRIPE)])


@functools.partial(
    pl.kernel,
    out_type=jax.ShapeDtypeStruct((2, NPAD), jnp.float32),
    mesh=_mesh,
    scratch_types=[
        pltpu.VMEM((WIN, CH), jnp.int32),
        pltpu.VMEM((WIN, CH), jnp.int32),
        pltpu.VMEM((CH,), jnp.float32),
        pltpu.VMEM((STRIPE,), jnp.float32),
        pltpu.VMEM_SHARED((NPAD,), jnp.float32),
        pltpu.SemaphoreType.DMA,
    ],
)
def _cw_kernel(ivd_hbm, src_hbm, dst_hbm, out_hbm,
               sidx_v, didx_v, wbuf, zero_v, acc, sem):
    cid = lax.axis_index("c")
    sid = lax.axis_index("s")
    wid = _wid()

    def fill_zero(i, carry):
        zero_v[pl.ds(i * 16, 16)] = jnp.zeros((16,), jnp.float32)
        return carry

    lax.fori_loop(0, STRIPE // 16, fill_zero, 0)
    pltpu.sync_copy(zero_v, acc.at[pl.ds(sid * STRIPE, STRIPE)])
    plsc.subcore_barrier()

    def window(w, carry):
        pltpu.sync_copy(src_hbm.at[wid, pl.ds(w * WIN, WIN)], sidx_v)
        pltpu.sync_copy(dst_hbm.at[wid, pl.ds(w * WIN, WIN)], didx_v)

        def chunk(j, carry2):
            pltpu.async_copy(ivd_hbm.at[didx_v.at[j]], wbuf, sem).wait()
            pltpu.sync_copy(wbuf, acc.at[sidx_v.at[j]], add=True)
            return carry2

        lax.fori_loop(0, WIN, chunk, 0)
        return carry

    lax.fori_loop(0, NCH // WIN, window, 0)
    plsc.subcore_barrier()
    pltpu.sync_copy(acc.at[pl.ds(sid * STRIPE, STRIPE)],
                    out_hbm.at[cid, pl.ds(sid * STRIPE, STRIPE)])


@functools.partial(
    pl.kernel,
    out_type=jax.ShapeDtypeStruct((2, NPAD, D), jnp.float32),
    mesh=_mesh,
    scratch_types=[
        pltpu.VMEM((WIN, CH), jnp.int32),
        pltpu.VMEM((WIN, CH), jnp.int32),
        pltpu.VMEM((CH, D), jnp.float32),
        pltpu.VMEM((CH, D), jnp.float32),
        pltpu.VMEM_SHARED((NPAD, D), jnp.float32),
        pltpu.SemaphoreType.DMA,
        pltpu.SemaphoreType.DMA,
    ],
)
def _agg_kernel(h_hbm, src_hbm, dst_hbm, out_hbm,
                sidx_v, didx_v, buf_a, buf_b, acc, sem_a, sem_b):
    cid = lax.axis_index("c")
    sid = lax.axis_index("s")
    wid = _wid()

    def fill_zero(i, carry):
        z = jnp.zeros((16,), jnp.float32)
        for c in range(D // 16):
            buf_a[i, pl.ds(c * 16, 16)] = z
        return carry

    lax.fori_loop(0, CH, fill_zero, 0)
    for k in range(STRIPE // CH):
        pltpu.sync_copy(buf_a, acc.at[pl.ds(sid * STRIPE + k * CH, CH)])
    plsc.subcore_barrier()

    def window(w, carry):
        pltpu.sync_copy(src_hbm.at[wid, pl.ds(w * WIN, WIN)], sidx_v)
        pltpu.sync_copy(dst_hbm.at[wid, pl.ds(w * WIN, WIN)], didx_v)

        def pair(j2, carry2):
            j = j2 * 2
            cp_a = pltpu.async_copy(h_hbm.at[sidx_v.at[j]], buf_a, sem_a)
            cp_b = pltpu.async_copy(h_hbm.at[sidx_v.at[j + 1]], buf_b, sem_b)
            cp_a.wait()
            pltpu.sync_copy(buf_a, acc.at[didx_v.at[j]], add=True)
            cp_b.wait()
            pltpu.sync_copy(buf_b, acc.at[didx_v.at[j + 1]], add=True)
            return carry2

        lax.fori_loop(0, WIN // 2, pair, 0)
        return carry

    lax.fori_loop(0, NCH // WIN, window, 0)
    plsc.subcore_barrier()
    pltpu.sync_copy(acc.at[pl.ds(sid * STRIPE, STRIPE)],
                    out_hbm.at[cid, pl.ds(sid * STRIPE, STRIPE)])


def _enc_body(x_ref, w_ref, b_ref, o_ref):
    o_ref[...] = jnp.dot(x_ref[...], w_ref[...],
                         preferred_element_type=jnp.float32) + b_ref[...]


def _glue_body(dp_ref, o_ref):
    o_ref[...] = 1.0 / jnp.maximum(dp_ref[0] + dp_ref[1], 1.0)


def _comb_body(h_ref, ap_ref, iv_ref, wr_ref, wn_ref, b_ref, o_ref):
    a = (ap_ref[0] + ap_ref[1]) * iv_ref[...]
    z = (jnp.dot(h_ref[...], wr_ref[...], preferred_element_type=jnp.float32)
         + jnp.dot(a, wn_ref[...], preferred_element_type=jnp.float32)
         + b_ref[...])
    o_ref[...] = jnp.maximum(z, 0.0)


def _final_body(h_ref, ap_ref, iv_ref, cp_ref,
                wr1_ref, wn1_ref, b1_ref,
                wr2_ref, wn2_ref, b2_ref,
                wd_ref, bd_ref, o_ref, acc_ref):
    i = pl.program_id(0)
    a = (ap_ref[0] + ap_ref[1]) * iv_ref[...]
    h2 = jnp.maximum(
        jnp.dot(h_ref[...], wr1_ref[...], preferred_element_type=jnp.float32)
        + jnp.dot(a, wn1_ref[...], preferred_element_type=jnp.float32)
        + b1_ref[...], 0.0)
    rows = lax.broadcasted_iota(jnp.int32, (BLK, 1), 0) + i * BLK
    m = (rows < N).astype(jnp.float32)
    c = (cp_ref[0] + cp_ref[1]) * m
    s0 = jnp.sum(h2 * m, axis=0, keepdims=True)
    s1 = jnp.sum(h2 * c, axis=0, keepdims=True)

    @pl.when(i == 0)
    def _():
        acc_ref[...] = jnp.zeros((8, D), jnp.float32)

    acc_ref[0:1, :] += s0
    acc_ref[1:2, :] += s1

    @pl.when(i == GRID - 1)
    def _():
        inv_n = 1.0 / N
        g = (jnp.dot(acc_ref[0:1, :] * inv_n, wr2_ref[...],
                     preferred_element_type=jnp.float32)
             + jnp.dot(acc_ref[1:2, :] * inv_n, wn2_ref[...],
                       preferred_element_type=jnp.float32)
             + b2_ref[...])
        logits = jnp.dot(g, wd_ref[...],
                         preferred_element_type=jnp.float32) + bd_ref[...]
        z = logits - jnp.max(logits, axis=-1, keepdims=True)
        ez = jnp.exp(z)
        o_ref[...] = ez / jnp.sum(ez, axis=-1, keepdims=True)


def _row_spec(width):
    return pl.BlockSpec((BLK, width), lambda i: (i, 0))


def _pair_spec(width):
    return pl.BlockSpec((2, BLK, width), lambda i: (0, i, 0))


def _full_spec(rows, cols):
    return pl.BlockSpec((rows, cols), lambda i: (0, 0))


def kernel(x, edge_index, W_enc, b_enc, Wr0, Wn0, b0, Wr1, Wn1, b1,
           Wr2, Wn2, b2, W_dec, b_dec):
    src = edge_index[0]
    dst = edge_index[1]
    pad = N + (jnp.arange(EPAD - E, dtype=jnp.int32) % (NPAD - N))
    srcp = jnp.concatenate([src, pad]).reshape(NW, NCH, CH)
    dstp = jnp.concatenate([dst, pad]).reshape(NW, NCH, CH)
    x_pad = jnp.pad(x, ((0, NPAD - N), (0, 0)))
    b_enc2 = b_enc.reshape(1, D)
    b0_2 = b0.reshape(1, D)
    b1_2 = b1.reshape(1, D)
    b2_2 = b2.reshape(1, D)
    b_dec2 = b_dec.reshape(1, b_dec.shape[0])

    h0 = pl.pallas_call(
        _enc_body,
        grid=(GRID,),
        in_specs=[_row_spec(D), _full_spec(D, D), _full_spec(1, D)],
        out_specs=_row_spec(D),
        out_shape=jax.ShapeDtypeStruct((NPAD, D), jnp.float32),
    )(x_pad, W_enc, b_enc2)

    degp = _deg_kernel(dstp)
    ivd_col = pl.pallas_call(
        _glue_body,
        grid=(GRID,),
        in_specs=[_pair_spec(1)],
        out_specs=_row_spec(1),
        out_shape=jax.ShapeDtypeStruct((NPAD, 1), jnp.float32),
    )(degp.reshape(2, NPAD, 1))
    cp = _cw_kernel(ivd_col.reshape(NPAD), srcp, dstp)

    agg0 = _agg_kernel(h0, srcp, dstp)
    h1 = pl.pallas_call(
        _comb_body,
        grid=(GRID,),
        in_specs=[_row_spec(D), _pair_spec(D), _row_spec(1),
                  _full_spec(D, D), _full_spec(D, D), _full_spec(1, D)],
        out_specs=_row_spec(D),
        out_shape=jax.ShapeDtypeStruct((NPAD, D), jnp.float32),
    )(h0, agg0, ivd_col, Wr0, Wn0, b0_2)

    agg1 = _agg_kernel(h1, srcp, dstp)
    out_dim = b_dec.shape[0]
    out = pl.pallas_call(
        _final_body,
        grid=(GRID,),
        in_specs=[_row_spec(D), _pair_spec(D), _row_spec(1), _pair_spec(1),
                  _full_spec(D, D), _full_spec(D, D), _full_spec(1, D),
                  _full_spec(D, D), _full_spec(D, D), _full_spec(1, D),
                  _full_spec(D, out_dim), _full_spec(1, out_dim)],
        out_specs=pl.BlockSpec((1, out_dim), lambda i: (0, 0)),
        out_shape=jax.ShapeDtypeStruct((1, out_dim), jnp.float32),
        scratch_shapes=[pltpu.VMEM((8, D), jnp.float32)],
    )(h1, agg1, ivd_col, cp.reshape(2, NPAD, 1),
      Wr1, Wn1, b1_2, Wr2, Wn2, b2_2, W_dec, b_dec2)
    return out
